# Initial kernel scaffold; baseline (speedup 1.0000x reference)
import jax
import jax.numpy as jnp
from jax.experimental import pallas as pl


def kernel(x, edge_index, W, att_src, att_dst, bias):
    raise NotImplementedError("write your pallas kernel here")


# speedup vs baseline: 18.6338x; 18.6338x over previous
import functools

import jax
import jax.numpy as jnp
from jax import lax
from jax.experimental import pallas as pl
from jax.experimental.pallas import tpu as pltpu
from jax.experimental.pallas import tpu_sc as plsc

N = 10000
D = 128
C = 128
E_RAW = 320000
E1 = E_RAW + N
NC = 2
NS = 16
NW = NC * NS
K = 64
NB = 162
E_PAD = NW * K * NB
PER_TILE = K * NB
N_PAD = 10240
ROWS_PER_TILE = N_PAD // NS


def _lrelu(v):
    return jnp.where(v > 0, v, 0.2 * v)


_BN1 = 400


def _stage1_body(x_ref, w_ref, atts_ref, attd_ref,
                 h_ref, asrc_ref, adst_ref, mx_ref):
    i = pl.program_id(0)
    h = jnp.dot(x_ref[...], w_ref[...], preferred_element_type=jnp.float32)
    h_ref[...] = h
    asv = jnp.sum(h * atts_ref[...], axis=1, keepdims=True)
    adv = jnp.sum(h * attd_ref[...], axis=1, keepdims=True)
    asrc_ref[...] = asv
    adst_ref[...] = adv
    bm = jnp.max(asv)

    @pl.when(i == 0)
    def _():
        mx_ref[...] = jnp.full((1, 16), -jnp.inf, jnp.float32)

    mx_ref[...] = jnp.maximum(mx_ref[...], bm)


def _stage1(x, W, att_src, att_dst):
    grid = (N // _BN1,)
    return pl.pallas_call(
        _stage1_body,
        grid=grid,
        in_specs=[
            pl.BlockSpec((_BN1, D), lambda i: (i, 0)),
            pl.BlockSpec((D, C), lambda i: (0, 0)),
            pl.BlockSpec((1, C), lambda i: (0, 0)),
            pl.BlockSpec((1, C), lambda i: (0, 0)),
        ],
        out_specs=[
            pl.BlockSpec((_BN1, C), lambda i: (i, 0)),
            pl.BlockSpec((_BN1, 1), lambda i: (i, 0)),
            pl.BlockSpec((_BN1, 1), lambda i: (i, 0)),
            pl.BlockSpec((1, 16), lambda i: (0, 0)),
        ],
        out_shape=[
            jax.ShapeDtypeStruct((N, C), jnp.float32),
            jax.ShapeDtypeStruct((N, 1), jnp.float32),
            jax.ShapeDtypeStruct((N, 1), jnp.float32),
            jax.ShapeDtypeStruct((1, 16), jnp.float32),
        ],
    )(x, W, att_src, att_dst)


def _stage2_body(h_hbm, src_hbm, dst_hbm, asrc_hbm, adst_hbm, mx_hbm,
                 msg_hbm, sw_hbm,
                 asrc_v, adst_v, src_v, dst_v, rows_v, w_buf, zs_v,
                 mx_v, out_sh, s_sh, sem):
    c = lax.axis_index("c")
    s = lax.axis_index("s")
    wid = s * NC + c
    base = wid * PER_TILE
    r0 = s * ROWS_PER_TILE

    zv = jnp.zeros((16,), jnp.float32)

    def zrow(k, _):
        for j in range(C // 16):
            rows_v[k, pl.ds(j * 16, 16)] = zv
        return 0

    lax.fori_loop(0, K, zrow, 0)

    def zs(k, _):
        zs_v[pl.ds(k * 16, 16)] = zv
        return 0

    lax.fori_loop(0, ROWS_PER_TILE // 16, zs, 0)
    for i in range(ROWS_PER_TILE // K):
        pltpu.sync_copy(rows_v, out_sh.at[pl.ds(r0 + i * K, K)])
    pltpu.sync_copy(zs_v, s_sh.at[pl.ds(r0, ROWS_PER_TILE)])

    pltpu.sync_copy(asrc_hbm, asrc_v)
    pltpu.sync_copy(adst_hbm, adst_v)
    pltpu.sync_copy(mx_hbm, mx_v)
    plsc.subcore_barrier()

    mx = mx_v[...]
    lane = lax.iota(jnp.int32, 16)

    def batch(b, _):
        off = base + b * K
        pltpu.sync_copy(src_hbm.at[pl.ds(off, K)], src_v)
        pltpu.sync_copy(dst_hbm.at[pl.ds(off, K)], dst_v)
        pltpu.async_copy(h_hbm.at[src_v], rows_v, sem).wait()
        for g in range(K // 16):
            sidx = src_v[pl.ds(g * 16, 16)]
            didx = dst_v[pl.ds(g * 16, 16)]
            a_s = plsc.load_gather(asrc_v, [sidx])
            a_d = plsc.load_gather(adst_v, [didx])
            e = _lrelu(a_s + a_d)
            bnd = _lrelu(mx + a_d)
            w = jnp.exp(e - bnd)
            eid = off + g * 16 + lane
            w = jnp.where(eid < E1, w, 0.0)
            w_buf[pl.ds(g * 16, 16)] = w

        def scale(k, _):
            wv = plsc.load_gather(w_buf, [jnp.full((16,), k, jnp.int32)])
            for j in range(C // 16):
                rows_v[k, pl.ds(j * 16, 16)] = (
                    rows_v[k, pl.ds(j * 16, 16)] * wv)
            return 0

        lax.fori_loop(0, K, scale, 0)

        pltpu.async_copy(rows_v, out_sh.at[dst_v], sem, add=True).wait()
        pltpu.async_copy(w_buf, s_sh.at[dst_v], sem, add=True).wait()
        return 0

    lax.fori_loop(0, NB, batch, 0)
    plsc.subcore_barrier()

    pltpu.sync_copy(out_sh.at[pl.ds(r0, ROWS_PER_TILE)],
                    msg_hbm.at[c, pl.ds(r0, ROWS_PER_TILE)])
    pltpu.sync_copy(s_sh.at[pl.ds(r0, ROWS_PER_TILE)],
                    sw_hbm.at[pl.ds(c * N_PAD + r0, ROWS_PER_TILE)])


def _stage2(h, src_p, dst_p, a_src, a_dst, mx):
    mesh = plsc.VectorSubcoreMesh(core_axis_name="c", subcore_axis_name="s")
    kern = functools.partial(
        pl.kernel,
        mesh=mesh,
        compiler_params=pltpu.CompilerParams(
            needs_layout_passes=False,
            use_tc_tiling_on_sc=False,
        ),
        out_type=[
            jax.ShapeDtypeStruct((NC, N_PAD, C), jnp.float32),
            jax.ShapeDtypeStruct((NC * N_PAD,), jnp.float32),
        ],
        scratch_types=[
            pltpu.VMEM((N,), jnp.float32),
            pltpu.VMEM((N,), jnp.float32),
            pltpu.VMEM((K,), jnp.int32),
            pltpu.VMEM((K,), jnp.int32),
            pltpu.VMEM((K, C), jnp.float32),
            pltpu.VMEM((K,), jnp.float32),
            pltpu.VMEM((ROWS_PER_TILE,), jnp.float32),
            pltpu.VMEM((16,), jnp.float32),
            pltpu.VMEM_SHARED((N_PAD, C), jnp.float32),
            pltpu.VMEM_SHARED((N_PAD,), jnp.float32),
            pltpu.SemaphoreType.DMA,
        ],
    )(_stage2_body)
    return kern(h, src_p, dst_p, a_src, a_dst, mx)


_BN3 = 400


def _stage3_body(msg_ref, sw_ref, bias_ref, out_ref):
    m = msg_ref[0] + msg_ref[1]
    s = sw_ref[0] + sw_ref[1]
    o = m / (s + 1e-16) + bias_ref[...]
    o = jnp.maximum(o, 0.0)
    nrm = jnp.sqrt(jnp.sum(o * o, axis=1, keepdims=True))
    nrm = jnp.maximum(nrm, 1e-12)
    out_ref[...] = o / nrm


def _stage3(msg_p, sw_p, bias):
    grid = (N // _BN3,)
    return pl.pallas_call(
        _stage3_body,
        grid=grid,
        in_specs=[
            pl.BlockSpec((NC, _BN3, C), lambda i: (0, i, 0)),
            pl.BlockSpec((NC, _BN3, 1), lambda i: (0, i, 0)),
            pl.BlockSpec((1, C), lambda i: (0, 0)),
        ],
        out_specs=pl.BlockSpec((_BN3, C), lambda i: (i, 0)),
        out_shape=jax.ShapeDtypeStruct((N, C), jnp.float32),
    )(msg_p, sw_p, bias)


def kernel(x, edge_index, W, att_src, att_dst, bias):
    h, a_src, a_dst, mx = _stage1(x, W, att_src, att_dst)

    loop = jnp.arange(N, dtype=edge_index.dtype)
    src = jnp.concatenate([edge_index[0], loop]).astype(jnp.int32)
    dst = jnp.concatenate([edge_index[1], loop]).astype(jnp.int32)
    src_p = jnp.pad(src, (0, E_PAD - E1))
    dst_p = jnp.pad(dst, (0, E_PAD - E1))

    msg_p, sw_flat = _stage2(h, src_p, dst_p,
                             a_src.reshape(N), a_dst.reshape(N),
                             mx.reshape(16))
    sw_p = sw_flat.reshape(NC, N_PAD, 1)

    return _stage3(msg_p, sw_p, bias.reshape(1, C))


# speedup vs baseline: 26.2991x; 1.4114x over previous
import functools

import jax
import jax.numpy as jnp
from jax import lax
from jax.experimental import pallas as pl
from jax.experimental.pallas import tpu as pltpu
from jax.experimental.pallas import tpu_sc as plsc

N = 10000
D = 128
C = 128
E_RAW = 320000
E1 = E_RAW + N
NC = 2
NS = 16
NW = NC * NS
K = 64
NB = 162
E_PAD = NW * K * NB
PER_TILE = K * NB
N_PAD = 10240
ROWS_PER_TILE = N_PAD // NS


def _lrelu(v):
    return jnp.where(v > 0, v, 0.2 * v)


_BN1 = 400


def _stage1_body(x_ref, w_ref, atts_ref, attd_ref,
                 h_ref, asrc_ref, adst_ref, mx_ref):
    i = pl.program_id(0)
    h = jnp.dot(x_ref[...], w_ref[...], preferred_element_type=jnp.float32)
    h_ref[...] = h
    asv = jnp.sum(h * atts_ref[...], axis=1, keepdims=True)
    adv = jnp.sum(h * attd_ref[...], axis=1, keepdims=True)
    asrc_ref[...] = asv
    adst_ref[...] = adv
    bm = jnp.max(asv)

    @pl.when(i == 0)
    def _():
        mx_ref[...] = jnp.full((1, 16), -jnp.inf, jnp.float32)

    mx_ref[...] = jnp.maximum(mx_ref[...], bm)


def _stage1(x, W, att_src, att_dst):
    grid = (N // _BN1,)
    return pl.pallas_call(
        _stage1_body,
        grid=grid,
        in_specs=[
            pl.BlockSpec((_BN1, D), lambda i: (i, 0)),
            pl.BlockSpec((D, C), lambda i: (0, 0)),
            pl.BlockSpec((1, C), lambda i: (0, 0)),
            pl.BlockSpec((1, C), lambda i: (0, 0)),
        ],
        out_specs=[
            pl.BlockSpec((_BN1, C), lambda i: (i, 0)),
            pl.BlockSpec((_BN1, 1), lambda i: (i, 0)),
            pl.BlockSpec((_BN1, 1), lambda i: (i, 0)),
            pl.BlockSpec((1, 16), lambda i: (0, 0)),
        ],
        out_shape=[
            jax.ShapeDtypeStruct((N, C), jnp.float32),
            jax.ShapeDtypeStruct((N, 1), jnp.float32),
            jax.ShapeDtypeStruct((N, 1), jnp.float32),
            jax.ShapeDtypeStruct((1, 16), jnp.float32),
        ],
    )(x, W, att_src, att_dst)


def _stage2_body(h_hbm, src_hbm, dst_hbm, asrc_hbm, adst_hbm, mx_hbm,
                 msg_hbm, sw_hbm,
                 asrc_v, adst_v, src0_v, dst0_v, src1_v, dst1_v,
                 rows0_v, rows1_v, w0_buf, w1_buf, zs_v, mx_v,
                 out_sh, s_sh, sem_i, sem_g0, sem_g1, sem_s):
    c = lax.axis_index("c")
    s = lax.axis_index("s")
    wid = s * NC + c
    base = wid * PER_TILE
    r0 = s * ROWS_PER_TILE

    zv = jnp.zeros((16,), jnp.float32)

    def zrow(k, _):
        for j in range(C // 16):
            rows0_v[k, pl.ds(j * 16, 16)] = zv
        return 0

    lax.fori_loop(0, K, zrow, 0)

    def zs(k, _):
        zs_v[pl.ds(k * 16, 16)] = zv
        return 0

    lax.fori_loop(0, ROWS_PER_TILE // 16, zs, 0)
    for i in range(ROWS_PER_TILE // K):
        pltpu.sync_copy(rows0_v, out_sh.at[pl.ds(r0 + i * K, K)])
    pltpu.sync_copy(zs_v, s_sh.at[pl.ds(r0, ROWS_PER_TILE)])

    pltpu.sync_copy(asrc_hbm, asrc_v)
    pltpu.sync_copy(adst_hbm, adst_v)
    pltpu.sync_copy(mx_hbm, mx_v)
    plsc.subcore_barrier()

    mx = mx_v[...]
    lane = lax.iota(jnp.int32, 16)

    def weights(off, src_v, dst_v, w_buf):
        for g in range(K // 16):
            sidx = src_v[pl.ds(g * 16, 16)]
            didx = dst_v[pl.ds(g * 16, 16)]
            a_s = plsc.load_gather(asrc_v, [sidx])
            a_d = plsc.load_gather(adst_v, [didx])
            e = _lrelu(a_s + a_d)
            bnd = _lrelu(mx + a_d)
            w = jnp.exp(e - bnd)
            eid = off + g * 16 + lane
            w = jnp.where(eid < E1, w, 0.0)
            w_buf[pl.ds(g * 16, 16)] = w

    def scale(rows_v, w_buf):
        def body(k, _):
            wv = plsc.load_gather(w_buf, [jnp.full((16,), k, jnp.int32)])
            for j in range(C // 16):
                rows_v[k, pl.ds(j * 16, 16)] = (
                    rows_v[k, pl.ds(j * 16, 16)] * wv)
            return 0

        lax.fori_loop(0, K, body, 0)

    def pair(p, _):
        off0 = base + (2 * p) * K
        off1 = off0 + K
        pltpu.async_copy(src_hbm.at[pl.ds(off0, K)], src0_v, sem_i)
        pltpu.async_copy(dst_hbm.at[pl.ds(off0, K)], dst0_v, sem_i)
        pltpu.async_copy(src_hbm.at[pl.ds(off1, K)], src1_v, sem_i)
        pltpu.async_copy(dst_hbm.at[pl.ds(off1, K)], dst1_v, sem_i)
        pltpu.make_async_copy(src_hbm.at[pl.ds(off0, K)], src0_v, sem_i).wait()
        pltpu.make_async_copy(dst_hbm.at[pl.ds(off0, K)], dst0_v, sem_i).wait()
        pltpu.make_async_copy(src_hbm.at[pl.ds(off1, K)], src1_v, sem_i).wait()
        pltpu.make_async_copy(dst_hbm.at[pl.ds(off1, K)], dst1_v, sem_i).wait()
        g0 = pltpu.async_copy(h_hbm.at[src0_v], rows0_v, sem_g0)
        g1 = pltpu.async_copy(h_hbm.at[src1_v], rows1_v, sem_g1)
        weights(off0, src0_v, dst0_v, w0_buf)
        weights(off1, src1_v, dst1_v, w1_buf)
        g0.wait()
        scale(rows0_v, w0_buf)
        s0a = pltpu.async_copy(rows0_v, out_sh.at[dst0_v], sem_s, add=True)
        s0b = pltpu.async_copy(w0_buf, s_sh.at[dst0_v], sem_s, add=True)
        g1.wait()
        scale(rows1_v, w1_buf)
        s1a = pltpu.async_copy(rows1_v, out_sh.at[dst1_v], sem_s, add=True)
        s1b = pltpu.async_copy(w1_buf, s_sh.at[dst1_v], sem_s, add=True)
        s0a.wait()
        s0b.wait()
        s1a.wait()
        s1b.wait()
        return 0

    lax.fori_loop(0, NB // 2, pair, 0)
    plsc.subcore_barrier()

    pltpu.sync_copy(out_sh.at[pl.ds(r0, ROWS_PER_TILE)],
                    msg_hbm.at[c, pl.ds(r0, ROWS_PER_TILE)])
    pltpu.sync_copy(s_sh.at[pl.ds(r0, ROWS_PER_TILE)],
                    sw_hbm.at[pl.ds(c * N_PAD + r0, ROWS_PER_TILE)])


def _stage2(h, src_p, dst_p, a_src, a_dst, mx):
    mesh = plsc.VectorSubcoreMesh(core_axis_name="c", subcore_axis_name="s")
    kern = functools.partial(
        pl.kernel,
        mesh=mesh,
        compiler_params=pltpu.CompilerParams(
            needs_layout_passes=False,
            use_tc_tiling_on_sc=False,
        ),
        out_type=[
            jax.ShapeDtypeStruct((NC, N_PAD, C), jnp.float32),
            jax.ShapeDtypeStruct((NC * N_PAD,), jnp.float32),
        ],
        scratch_types=[
            pltpu.VMEM((N,), jnp.float32),
            pltpu.VMEM((N,), jnp.float32),
            pltpu.VMEM((K,), jnp.int32),
            pltpu.VMEM((K,), jnp.int32),
            pltpu.VMEM((K,), jnp.int32),
            pltpu.VMEM((K,), jnp.int32),
            pltpu.VMEM((K, C), jnp.float32),
            pltpu.VMEM((K, C), jnp.float32),
            pltpu.VMEM((K,), jnp.float32),
            pltpu.VMEM((K,), jnp.float32),
            pltpu.VMEM((ROWS_PER_TILE,), jnp.float32),
            pltpu.VMEM((16,), jnp.float32),
            pltpu.VMEM_SHARED((N_PAD, C), jnp.float32),
            pltpu.VMEM_SHARED((N_PAD,), jnp.float32),
            pltpu.SemaphoreType.DMA,
            pltpu.SemaphoreType.DMA,
            pltpu.SemaphoreType.DMA,
            pltpu.SemaphoreType.DMA,
        ],
    )(_stage2_body)
    return kern(h, src_p, dst_p, a_src, a_dst, mx)


_BN3 = 400


def _stage3_body(msg_ref, sw_ref, bias_ref, out_ref):
    m = msg_ref[0] + msg_ref[1]
    s = sw_ref[0] + sw_ref[1]
    o = m / (s + 1e-16) + bias_ref[...]
    o = jnp.maximum(o, 0.0)
    nrm = jnp.sqrt(jnp.sum(o * o, axis=1, keepdims=True))
    nrm = jnp.maximum(nrm, 1e-12)
    out_ref[...] = o / nrm


def _stage3(msg_p, sw_p, bias):
    grid = (N // _BN3,)
    return pl.pallas_call(
        _stage3_body,
        grid=grid,
        in_specs=[
            pl.BlockSpec((NC, _BN3, C), lambda i: (0, i, 0)),
            pl.BlockSpec((NC, _BN3, 1), lambda i: (0, i, 0)),
            pl.BlockSpec((1, C), lambda i: (0, 0)),
        ],
        out_specs=pl.BlockSpec((_BN3, C), lambda i: (i, 0)),
        out_shape=jax.ShapeDtypeStruct((N, C), jnp.float32),
    )(msg_p, sw_p, bias)


def kernel(x, edge_index, W, att_src, att_dst, bias):
    h, a_src, a_dst, mx = _stage1(x, W, att_src, att_dst)

    loop = jnp.arange(N, dtype=edge_index.dtype)
    src = jnp.concatenate([edge_index[0], loop]).astype(jnp.int32)
    dst = jnp.concatenate([edge_index[1], loop]).astype(jnp.int32)
    src_p = jnp.pad(src, (0, E_PAD - E1))
    dst_p = jnp.pad(dst, (0, E_PAD - E1))

    msg_p, sw_flat = _stage2(h, src_p, dst_p,
                             a_src.reshape(N), a_dst.reshape(N),
                             mx.reshape(16))
    sw_p = sw_flat.reshape(NC, N_PAD, 1)

    return _stage3(msg_p, sw_p, bias.reshape(1, C))


# speedup vs baseline: 29.1960x; 1.1101x over previous
import functools

import jax
import jax.numpy as jnp
from jax import lax
from jax.experimental import pallas as pl
from jax.experimental.pallas import tpu as pltpu
from jax.experimental.pallas import tpu_sc as plsc

N = 10000
D = 128
C = 128
E_RAW = 320000
E1 = E_RAW + N
NC = 2
NS = 16
NW = NC * NS
K = 64
NB = 162
E_PAD = NW * K * NB
PER_TILE = K * NB
N_PAD = 10240
ROWS_PER_TILE = N_PAD // NS


def _lrelu(v):
    return jnp.where(v > 0, v, 0.2 * v)


_BN1 = 400


def _stage1_body(x_ref, w_ref, atts_ref, attd_ref,
                 h_ref, asrc_ref, adst_ref, mx_ref):
    i = pl.program_id(0)
    h = jnp.dot(x_ref[...], w_ref[...], preferred_element_type=jnp.float32)
    h_ref[...] = h
    asv = jnp.sum(h * atts_ref[...], axis=1, keepdims=True)
    adv = jnp.sum(h * attd_ref[...], axis=1, keepdims=True)
    asrc_ref[...] = asv
    adst_ref[...] = adv
    bm = jnp.max(asv)

    @pl.when(i == 0)
    def _():
        mx_ref[...] = jnp.full((1, 16), -jnp.inf, jnp.float32)

    mx_ref[...] = jnp.maximum(mx_ref[...], bm)


def _stage1(x, W, att_src, att_dst):
    grid = (N // _BN1,)
    return pl.pallas_call(
        _stage1_body,
        grid=grid,
        in_specs=[
            pl.BlockSpec((_BN1, D), lambda i: (i, 0)),
            pl.BlockSpec((D, C), lambda i: (0, 0)),
            pl.BlockSpec((1, C), lambda i: (0, 0)),
            pl.BlockSpec((1, C), lambda i: (0, 0)),
        ],
        out_specs=[
            pl.BlockSpec((_BN1, C), lambda i: (i, 0)),
            pl.BlockSpec((_BN1, 1), lambda i: (i, 0)),
            pl.BlockSpec((_BN1, 1), lambda i: (i, 0)),
            pl.BlockSpec((1, 16), lambda i: (0, 0)),
        ],
        out_shape=[
            jax.ShapeDtypeStruct((N, C), jnp.float32),
            jax.ShapeDtypeStruct((N, 1), jnp.float32),
            jax.ShapeDtypeStruct((N, 1), jnp.float32),
            jax.ShapeDtypeStruct((1, 16), jnp.float32),
        ],
    )(x, W, att_src, att_dst)


def _stage2_body(h_hbm, src_hbm, dst_hbm, asrc_hbm, adst_hbm, mx_hbm,
                 msg_hbm, sw_hbm,
                 asrc_v, adst_v, src0_v, dst0_v, src1_v, dst1_v,
                 rows0_v, rows1_v, w0_buf, w1_buf, zs_v, mx_v,
                 out_sh, s_sh, sem_i, sem_g0, sem_g1, sem_s):
    c = lax.axis_index("c")
    s = lax.axis_index("s")
    wid = s * NC + c
    base = wid * PER_TILE
    r0 = s * ROWS_PER_TILE

    zv = jnp.zeros((16,), jnp.float32)

    def zrow(k, _):
        for j in range(C // 16):
            rows0_v[k, pl.ds(j * 16, 16)] = zv
        return 0

    lax.fori_loop(0, K, zrow, 0)

    def zs(k, _):
        zs_v[pl.ds(k * 16, 16)] = zv
        return 0

    lax.fori_loop(0, ROWS_PER_TILE // 16, zs, 0)
    for i in range(ROWS_PER_TILE // K):
        pltpu.sync_copy(rows0_v, out_sh.at[pl.ds(r0 + i * K, K)])
    pltpu.sync_copy(zs_v, s_sh.at[pl.ds(r0, ROWS_PER_TILE)])

    pltpu.sync_copy(asrc_hbm, asrc_v)
    pltpu.sync_copy(adst_hbm, adst_v)
    pltpu.sync_copy(mx_hbm, mx_v)
    plsc.subcore_barrier()

    mx = mx_v[...]
    lane = lax.iota(jnp.int32, 16)

    def weights(off, src_v, dst_v, w_buf):
        for g in range(K // 16):
            sidx = src_v[pl.ds(g * 16, 16)]
            didx = dst_v[pl.ds(g * 16, 16)]
            a_s = plsc.load_gather(asrc_v, [sidx])
            a_d = plsc.load_gather(adst_v, [didx])
            e = _lrelu(a_s + a_d)
            bnd = _lrelu(mx + a_d)
            w = jnp.exp(e - bnd)
            eid = off + g * 16 + lane
            w = jnp.where(eid < E1, w, 0.0)
            w_buf[pl.ds(g * 16, 16)] = w

    def scale(rows_v, w_buf):
        @plsc.parallel_loop(0, K, unroll=4)
        def body(k):
            wv = plsc.load_gather(w_buf, [jnp.full((16,), k, jnp.int32)])
            for j in range(C // 16):
                rows_v[k, pl.ds(j * 16, 16)] = (
                    rows_v[k, pl.ds(j * 16, 16)] * wv)

    def pair(p, _):
        off0 = base + (2 * p) * K
        off1 = off0 + K
        pltpu.async_copy(src_hbm.at[pl.ds(off0, K)], src0_v, sem_i)
        pltpu.async_copy(dst_hbm.at[pl.ds(off0, K)], dst0_v, sem_i)
        pltpu.async_copy(src_hbm.at[pl.ds(off1, K)], src1_v, sem_i)
        pltpu.async_copy(dst_hbm.at[pl.ds(off1, K)], dst1_v, sem_i)
        pltpu.make_async_copy(src_hbm.at[pl.ds(off0, K)], src0_v, sem_i).wait()
        pltpu.make_async_copy(dst_hbm.at[pl.ds(off0, K)], dst0_v, sem_i).wait()
        pltpu.make_async_copy(src_hbm.at[pl.ds(off1, K)], src1_v, sem_i).wait()
        pltpu.make_async_copy(dst_hbm.at[pl.ds(off1, K)], dst1_v, sem_i).wait()
        g0 = pltpu.async_copy(h_hbm.at[src0_v], rows0_v, sem_g0)
        g1 = pltpu.async_copy(h_hbm.at[src1_v], rows1_v, sem_g1)
        weights(off0, src0_v, dst0_v, w0_buf)
        weights(off1, src1_v, dst1_v, w1_buf)
        g0.wait()
        scale(rows0_v, w0_buf)
        s0a = pltpu.async_copy(rows0_v, out_sh.at[dst0_v], sem_s, add=True)
        s0b = pltpu.async_copy(w0_buf, s_sh.at[dst0_v], sem_s, add=True)
        g1.wait()
        scale(rows1_v, w1_buf)
        s1a = pltpu.async_copy(rows1_v, out_sh.at[dst1_v], sem_s, add=True)
        s1b = pltpu.async_copy(w1_buf, s_sh.at[dst1_v], sem_s, add=True)
        s0a.wait()
        s0b.wait()
        s1a.wait()
        s1b.wait()
        return 0

    lax.fori_loop(0, NB // 2, pair, 0)
    plsc.subcore_barrier()

    pltpu.sync_copy(out_sh.at[pl.ds(r0, ROWS_PER_TILE)],
                    msg_hbm.at[c, pl.ds(r0, ROWS_PER_TILE)])
    pltpu.sync_copy(s_sh.at[pl.ds(r0, ROWS_PER_TILE)],
                    sw_hbm.at[pl.ds(c * N_PAD + r0, ROWS_PER_TILE)])


def _stage2(h, src_p, dst_p, a_src, a_dst, mx):
    mesh = plsc.VectorSubcoreMesh(core_axis_name="c", subcore_axis_name="s")
    kern = functools.partial(
        pl.kernel,
        mesh=mesh,
        compiler_params=pltpu.CompilerParams(
            needs_layout_passes=False,
            use_tc_tiling_on_sc=False,
        ),
        out_type=[
            jax.ShapeDtypeStruct((NC, N_PAD, C), jnp.float32),
            jax.ShapeDtypeStruct((NC * N_PAD,), jnp.float32),
        ],
        scratch_types=[
            pltpu.VMEM((N,), jnp.float32),
            pltpu.VMEM((N,), jnp.float32),
            pltpu.VMEM((K,), jnp.int32),
            pltpu.VMEM((K,), jnp.int32),
            pltpu.VMEM((K,), jnp.int32),
            pltpu.VMEM((K,), jnp.int32),
            pltpu.VMEM((K, C), jnp.float32),
            pltpu.VMEM((K, C), jnp.float32),
            pltpu.VMEM((K,), jnp.float32),
            pltpu.VMEM((K,), jnp.float32),
            pltpu.VMEM((ROWS_PER_TILE,), jnp.float32),
            pltpu.VMEM((16,), jnp.float32),
            pltpu.VMEM_SHARED((N_PAD, C), jnp.float32),
            pltpu.VMEM_SHARED((N_PAD,), jnp.float32),
            pltpu.SemaphoreType.DMA,
            pltpu.SemaphoreType.DMA,
            pltpu.SemaphoreType.DMA,
            pltpu.SemaphoreType.DMA,
        ],
    )(_stage2_body)
    return kern(h, src_p, dst_p, a_src, a_dst, mx)


_BN3 = 400


def _stage3_body(msg_ref, sw_ref, bias_ref, out_ref):
    m = msg_ref[0] + msg_ref[1]
    s = sw_ref[0] + sw_ref[1]
    o = m / (s + 1e-16) + bias_ref[...]
    o = jnp.maximum(o, 0.0)
    nrm = jnp.sqrt(jnp.sum(o * o, axis=1, keepdims=True))
    nrm = jnp.maximum(nrm, 1e-12)
    out_ref[...] = o / nrm


def _stage3(msg_p, sw_p, bias):
    grid = (N // _BN3,)
    return pl.pallas_call(
        _stage3_body,
        grid=grid,
        in_specs=[
            pl.BlockSpec((NC, _BN3, C), lambda i: (0, i, 0)),
            pl.BlockSpec((NC, _BN3, 1), lambda i: (0, i, 0)),
            pl.BlockSpec((1, C), lambda i: (0, 0)),
        ],
        out_specs=pl.BlockSpec((_BN3, C), lambda i: (i, 0)),
        out_shape=jax.ShapeDtypeStruct((N, C), jnp.float32),
    )(msg_p, sw_p, bias)


def kernel(x, edge_index, W, att_src, att_dst, bias):
    h, a_src, a_dst, mx = _stage1(x, W, att_src, att_dst)

    loop = jnp.arange(N, dtype=edge_index.dtype)
    src = jnp.concatenate([edge_index[0], loop]).astype(jnp.int32)
    dst = jnp.concatenate([edge_index[1], loop]).astype(jnp.int32)
    src_p = jnp.pad(src, (0, E_PAD - E1))
    dst_p = jnp.pad(dst, (0, E_PAD - E1))

    msg_p, sw_flat = _stage2(h, src_p, dst_p,
                             a_src.reshape(N), a_dst.reshape(N),
                             mx.reshape(16))
    sw_p = sw_flat.reshape(NC, N_PAD, 1)

    return _stage3(msg_p, sw_p, bias.reshape(1, C))


# speedup vs baseline: 30.8819x; 1.0577x over previous
import functools

import jax
import jax.numpy as jnp
from jax import lax
from jax.experimental import pallas as pl
from jax.experimental.pallas import tpu as pltpu
from jax.experimental.pallas import tpu_sc as plsc

N = 10000
D = 128
C = 128
E_RAW = 320000
E1 = E_RAW + N
NC = 2
NS = 16
NW = NC * NS
K = 96
NB = 108
E_PAD = NW * K * NB
PER_TILE = K * NB
N_PAD = 10240
ROWS_PER_TILE = N_PAD // NS


def _lrelu(v):
    return jnp.where(v > 0, v, 0.2 * v)


_BN1 = 400


def _stage1_body(x_ref, w_ref, atts_ref, attd_ref,
                 h_ref, asrc_ref, adst_ref, mx_ref):
    i = pl.program_id(0)
    h = jnp.dot(x_ref[...], w_ref[...], preferred_element_type=jnp.float32)
    h_ref[...] = h
    asv = jnp.sum(h * atts_ref[...], axis=1, keepdims=True)
    adv = jnp.sum(h * attd_ref[...], axis=1, keepdims=True)
    asrc_ref[...] = asv
    adst_ref[...] = adv
    bm = jnp.max(asv)

    @pl.when(i == 0)
    def _():
        mx_ref[...] = jnp.full((1, 16), -jnp.inf, jnp.float32)

    mx_ref[...] = jnp.maximum(mx_ref[...], bm)


def _stage1(x, W, att_src, att_dst):
    grid = (N // _BN1,)
    return pl.pallas_call(
        _stage1_body,
        grid=grid,
        in_specs=[
            pl.BlockSpec((_BN1, D), lambda i: (i, 0)),
            pl.BlockSpec((D, C), lambda i: (0, 0)),
            pl.BlockSpec((1, C), lambda i: (0, 0)),
            pl.BlockSpec((1, C), lambda i: (0, 0)),
        ],
        out_specs=[
            pl.BlockSpec((_BN1, C), lambda i: (i, 0)),
            pl.BlockSpec((_BN1, 1), lambda i: (i, 0)),
            pl.BlockSpec((_BN1, 1), lambda i: (i, 0)),
            pl.BlockSpec((1, 16), lambda i: (0, 0)),
        ],
        out_shape=[
            jax.ShapeDtypeStruct((N, C), jnp.float32),
            jax.ShapeDtypeStruct((N, 1), jnp.float32),
            jax.ShapeDtypeStruct((N, 1), jnp.float32),
            jax.ShapeDtypeStruct((1, 16), jnp.float32),
        ],
    )(x, W, att_src, att_dst)


def _stage2_body(h_hbm, src_hbm, dst_hbm, asrc_hbm, adst_hbm, mx_hbm,
                 msg_hbm, sw_hbm,
                 asrc_v, adst_v, src0_v, dst0_v, src1_v, dst1_v,
                 rows0_v, rows1_v, w0_buf, w1_buf, zs_v, mx_v,
                 out_sh, s_sh, sem_i, sem_g0, sem_g1, sem_s):
    c = lax.axis_index("c")
    s = lax.axis_index("s")
    wid = s * NC + c
    base = wid * PER_TILE
    r0 = s * ROWS_PER_TILE

    zv = jnp.zeros((16,), jnp.float32)

    def zrow(k, _):
        for j in range(C // 16):
            rows0_v[k, pl.ds(j * 16, 16)] = zv
        return 0

    lax.fori_loop(0, K, zrow, 0)

    def zs(k, _):
        zs_v[pl.ds(k * 16, 16)] = zv
        return 0

    lax.fori_loop(0, ROWS_PER_TILE // 16, zs, 0)
    _nfull = ROWS_PER_TILE // K
    _rem = ROWS_PER_TILE - _nfull * K
    for i in range(_nfull):
        pltpu.sync_copy(rows0_v, out_sh.at[pl.ds(r0 + i * K, K)])
    if _rem:
        pltpu.sync_copy(rows0_v.at[pl.ds(0, _rem)],
                        out_sh.at[pl.ds(r0 + _nfull * K, _rem)])
    pltpu.sync_copy(zs_v, s_sh.at[pl.ds(r0, ROWS_PER_TILE)])

    pltpu.sync_copy(asrc_hbm, asrc_v)
    pltpu.sync_copy(adst_hbm, adst_v)
    pltpu.sync_copy(mx_hbm, mx_v)
    plsc.subcore_barrier()

    mx = mx_v[...]
    lane = lax.iota(jnp.int32, 16)

    def weights(off, src_v, dst_v, w_buf):
        for g in range(K // 16):
            sidx = src_v[pl.ds(g * 16, 16)]
            didx = dst_v[pl.ds(g * 16, 16)]
            a_s = plsc.load_gather(asrc_v, [sidx])
            a_d = plsc.load_gather(adst_v, [didx])
            e = _lrelu(a_s + a_d)
            bnd = _lrelu(mx + a_d)
            w = jnp.exp(e - bnd)
            eid = off + g * 16 + lane
            w = jnp.where(eid < E1, w, 0.0)
            w_buf[pl.ds(g * 16, 16)] = w

    def scale(rows_v, w_buf):
        @plsc.parallel_loop(0, K, unroll=4)
        def body(k):
            wv = plsc.load_gather(w_buf, [jnp.full((16,), k, jnp.int32)])
            for j in range(C // 16):
                rows_v[k, pl.ds(j * 16, 16)] = (
                    rows_v[k, pl.ds(j * 16, 16)] * wv)

    def pair(p, _):
        off0 = base + (2 * p) * K
        off1 = off0 + K
        pltpu.async_copy(src_hbm.at[pl.ds(off0, K)], src0_v, sem_i)
        pltpu.async_copy(dst_hbm.at[pl.ds(off0, K)], dst0_v, sem_i)
        pltpu.async_copy(src_hbm.at[pl.ds(off1, K)], src1_v, sem_i)
        pltpu.async_copy(dst_hbm.at[pl.ds(off1, K)], dst1_v, sem_i)
        pltpu.make_async_copy(src_hbm.at[pl.ds(off0, K)], src0_v, sem_i).wait()
        pltpu.make_async_copy(dst_hbm.at[pl.ds(off0, K)], dst0_v, sem_i).wait()
        pltpu.make_async_copy(src_hbm.at[pl.ds(off1, K)], src1_v, sem_i).wait()
        pltpu.make_async_copy(dst_hbm.at[pl.ds(off1, K)], dst1_v, sem_i).wait()
        g0 = pltpu.async_copy(h_hbm.at[src0_v], rows0_v, sem_g0)
        g1 = pltpu.async_copy(h_hbm.at[src1_v], rows1_v, sem_g1)
        weights(off0, src0_v, dst0_v, w0_buf)
        weights(off1, src1_v, dst1_v, w1_buf)
        g0.wait()
        scale(rows0_v, w0_buf)
        s0a = pltpu.async_copy(rows0_v, out_sh.at[dst0_v], sem_s, add=True)
        s0b = pltpu.async_copy(w0_buf, s_sh.at[dst0_v], sem_s, add=True)
        g1.wait()
        scale(rows1_v, w1_buf)
        s1a = pltpu.async_copy(rows1_v, out_sh.at[dst1_v], sem_s, add=True)
        s1b = pltpu.async_copy(w1_buf, s_sh.at[dst1_v], sem_s, add=True)
        s0a.wait()
        s0b.wait()
        s1a.wait()
        s1b.wait()
        return 0

    lax.fori_loop(0, NB // 2, pair, 0)
    plsc.subcore_barrier()

    pltpu.sync_copy(out_sh.at[pl.ds(r0, ROWS_PER_TILE)],
                    msg_hbm.at[c, pl.ds(r0, ROWS_PER_TILE)])
    pltpu.sync_copy(s_sh.at[pl.ds(r0, ROWS_PER_TILE)],
                    sw_hbm.at[pl.ds(c * N_PAD + r0, ROWS_PER_TILE)])


def _stage2(h, src_p, dst_p, a_src, a_dst, mx):
    mesh = plsc.VectorSubcoreMesh(core_axis_name="c", subcore_axis_name="s")
    kern = functools.partial(
        pl.kernel,
        mesh=mesh,
        compiler_params=pltpu.CompilerParams(
            needs_layout_passes=False,
            use_tc_tiling_on_sc=False,
        ),
        out_type=[
            jax.ShapeDtypeStruct((NC, N_PAD, C), jnp.float32),
            jax.ShapeDtypeStruct((NC * N_PAD,), jnp.float32),
        ],
        scratch_types=[
            pltpu.VMEM((N,), jnp.float32),
            pltpu.VMEM((N,), jnp.float32),
            pltpu.VMEM((K,), jnp.int32),
            pltpu.VMEM((K,), jnp.int32),
            pltpu.VMEM((K,), jnp.int32),
            pltpu.VMEM((K,), jnp.int32),
            pltpu.VMEM((K, C), jnp.float32),
            pltpu.VMEM((K, C), jnp.float32),
            pltpu.VMEM((K,), jnp.float32),
            pltpu.VMEM((K,), jnp.float32),
            pltpu.VMEM((ROWS_PER_TILE,), jnp.float32),
            pltpu.VMEM((16,), jnp.float32),
            pltpu.VMEM_SHARED((N_PAD, C), jnp.float32),
            pltpu.VMEM_SHARED((N_PAD,), jnp.float32),
            pltpu.SemaphoreType.DMA,
            pltpu.SemaphoreType.DMA,
            pltpu.SemaphoreType.DMA,
            pltpu.SemaphoreType.DMA,
        ],
    )(_stage2_body)
    return kern(h, src_p, dst_p, a_src, a_dst, mx)


_BN3 = 400


def _stage3_body(msg_ref, sw_ref, bias_ref, out_ref):
    m = msg_ref[0] + msg_ref[1]
    s = sw_ref[0] + sw_ref[1]
    o = m / (s + 1e-16) + bias_ref[...]
    o = jnp.maximum(o, 0.0)
    nrm = jnp.sqrt(jnp.sum(o * o, axis=1, keepdims=True))
    nrm = jnp.maximum(nrm, 1e-12)
    out_ref[...] = o / nrm


def _stage3(msg_p, sw_p, bias):
    grid = (N // _BN3,)
    return pl.pallas_call(
        _stage3_body,
        grid=grid,
        in_specs=[
            pl.BlockSpec((NC, _BN3, C), lambda i: (0, i, 0)),
            pl.BlockSpec((NC, _BN3, 1), lambda i: (0, i, 0)),
            pl.BlockSpec((1, C), lambda i: (0, 0)),
        ],
        out_specs=pl.BlockSpec((_BN3, C), lambda i: (i, 0)),
        out_shape=jax.ShapeDtypeStruct((N, C), jnp.float32),
    )(msg_p, sw_p, bias)


def kernel(x, edge_index, W, att_src, att_dst, bias):
    h, a_src, a_dst, mx = _stage1(x, W, att_src, att_dst)

    loop = jnp.arange(N, dtype=edge_index.dtype)
    src = jnp.concatenate([edge_index[0], loop]).astype(jnp.int32)
    dst = jnp.concatenate([edge_index[1], loop]).astype(jnp.int32)
    src_p = jnp.pad(src, (0, E_PAD - E1))
    dst_p = jnp.pad(dst, (0, E_PAD - E1))

    msg_p, sw_flat = _stage2(h, src_p, dst_p,
                             a_src.reshape(N), a_dst.reshape(N),
                             mx.reshape(16))
    sw_p = sw_flat.reshape(NC, N_PAD, 1)

    return _stage3(msg_p, sw_p, bias.reshape(1, C))


# speedup vs baseline: 33.1565x; 1.0737x over previous
import functools

import jax
import jax.numpy as jnp
from jax import lax
from jax.experimental import pallas as pl
from jax.experimental.pallas import tpu as pltpu
from jax.experimental.pallas import tpu_sc as plsc

N = 10000
D = 128
C = 128
E_RAW = 320000
E1 = E_RAW + N
NC = 2
NS = 16
NW = NC * NS
K = 96
NB = 108
E_PAD = NW * K * NB
PER_TILE = K * NB
N_PAD = 10240
ROWS_PER_TILE = N_PAD // NS


def _lrelu(v):
    return jnp.where(v > 0, v, 0.2 * v)


_BN1 = 400


def _stage1_body(x_ref, w_ref, atts_ref, attd_ref,
                 h_ref, asrc_ref, adst_ref, mx_ref):
    i = pl.program_id(0)
    h = jnp.dot(x_ref[...], w_ref[...], preferred_element_type=jnp.float32)
    h_ref[...] = h
    asv = jnp.sum(h * atts_ref[...], axis=1, keepdims=True)
    adv = jnp.sum(h * attd_ref[...], axis=1, keepdims=True)
    asrc_ref[...] = asv
    adst_ref[...] = adv
    bm = jnp.max(asv)

    @pl.when(i == 0)
    def _():
        mx_ref[...] = jnp.full((1, 16), -jnp.inf, jnp.float32)

    mx_ref[...] = jnp.maximum(mx_ref[...], bm)


def _stage1(x, W, att_src, att_dst):
    grid = (N // _BN1,)
    return pl.pallas_call(
        _stage1_body,
        grid=grid,
        in_specs=[
            pl.BlockSpec((_BN1, D), lambda i: (i, 0)),
            pl.BlockSpec((D, C), lambda i: (0, 0)),
            pl.BlockSpec((1, C), lambda i: (0, 0)),
            pl.BlockSpec((1, C), lambda i: (0, 0)),
        ],
        out_specs=[
            pl.BlockSpec((_BN1, C), lambda i: (i, 0)),
            pl.BlockSpec((_BN1, 1), lambda i: (i, 0)),
            pl.BlockSpec((_BN1, 1), lambda i: (i, 0)),
            pl.BlockSpec((1, 16), lambda i: (0, 0)),
        ],
        out_shape=[
            jax.ShapeDtypeStruct((N, C), jnp.float32),
            jax.ShapeDtypeStruct((N, 1), jnp.float32),
            jax.ShapeDtypeStruct((N, 1), jnp.float32),
            jax.ShapeDtypeStruct((1, 16), jnp.float32),
        ],
    )(x, W, att_src, att_dst)


def _stage2_body(h_hbm, src_hbm, dst_hbm, asrc_hbm, adst_hbm, mx_hbm,
                 msg_hbm, sw_hbm,
                 asrc_v, adst_v, src0_v, dst0_v, src1_v, dst1_v,
                 src2_v, dst2_v, src3_v, dst3_v,
                 rows0_v, rows1_v, w0_buf, w1_buf, zs_v, mx_v,
                 out_sh, s_sh, sem_i, sem_g0, sem_g1, sem_s):
    c = lax.axis_index("c")
    s = lax.axis_index("s")
    wid = s * NC + c
    base = wid * PER_TILE
    r0 = s * ROWS_PER_TILE

    zv = jnp.zeros((16,), jnp.float32)

    def zrow(k, _):
        for j in range(C // 16):
            rows0_v[k, pl.ds(j * 16, 16)] = zv
        return 0

    lax.fori_loop(0, K, zrow, 0)

    def zs(k, _):
        zs_v[pl.ds(k * 16, 16)] = zv
        return 0

    lax.fori_loop(0, ROWS_PER_TILE // 16, zs, 0)
    _nfull = ROWS_PER_TILE // K
    _rem = ROWS_PER_TILE - _nfull * K
    for i in range(_nfull):
        pltpu.sync_copy(rows0_v, out_sh.at[pl.ds(r0 + i * K, K)])
    if _rem:
        pltpu.sync_copy(rows0_v.at[pl.ds(0, _rem)],
                        out_sh.at[pl.ds(r0 + _nfull * K, _rem)])
    pltpu.sync_copy(zs_v, s_sh.at[pl.ds(r0, ROWS_PER_TILE)])

    pltpu.sync_copy(asrc_hbm, asrc_v)
    pltpu.sync_copy(adst_hbm, adst_v)
    pltpu.sync_copy(mx_hbm, mx_v)
    plsc.subcore_barrier()

    mx = mx_v[...]
    lane = lax.iota(jnp.int32, 16)

    def weights(off, src_v, dst_v, w_buf):
        for g in range(K // 16):
            sidx = src_v[pl.ds(g * 16, 16)]
            didx = dst_v[pl.ds(g * 16, 16)]
            a_s = plsc.load_gather(asrc_v, [sidx])
            a_d = plsc.load_gather(adst_v, [didx])
            e = _lrelu(a_s + a_d)
            bnd = _lrelu(mx + a_d)
            w = jnp.exp(e - bnd)
            eid = off + g * 16 + lane
            w = jnp.where(eid < E1, w, 0.0)
            w_buf[pl.ds(g * 16, 16)] = w

    def scale(rows_v, w_buf):
        @plsc.parallel_loop(0, K, unroll=4)
        def body(k):
            wv = plsc.load_gather(w_buf, [jnp.full((16,), k, jnp.int32)])
            for j in range(C // 16):
                rows_v[k, pl.ds(j * 16, 16)] = (
                    rows_v[k, pl.ds(j * 16, 16)] * wv)

    def fire_idx(p, sv0, dv0, sv1, dv1):
        off0 = base + (2 * p) * K
        off1 = off0 + K
        pltpu.async_copy(src_hbm.at[pl.ds(off0, K)], sv0, sem_i)
        pltpu.async_copy(dst_hbm.at[pl.ds(off0, K)], dv0, sem_i)
        pltpu.async_copy(src_hbm.at[pl.ds(off1, K)], sv1, sem_i)
        pltpu.async_copy(dst_hbm.at[pl.ds(off1, K)], dv1, sem_i)

    def drain_idx(sv0, dv0, sv1, dv1):
        pltpu.make_async_copy(src_hbm.at[pl.ds(base, K)], sv0, sem_i).wait()
        pltpu.make_async_copy(src_hbm.at[pl.ds(base, K)], dv0, sem_i).wait()
        pltpu.make_async_copy(src_hbm.at[pl.ds(base, K)], sv1, sem_i).wait()
        pltpu.make_async_copy(src_hbm.at[pl.ds(base, K)], dv1, sem_i).wait()

    def pair(p, me, nxt):
        sv0, dv0, sv1, dv1 = me
        drain_idx(*me)
        fire_idx(p + 1, *nxt)
        off0 = base + (2 * p) * K
        off1 = off0 + K
        g0 = pltpu.async_copy(h_hbm.at[sv0], rows0_v, sem_g0)
        g1 = pltpu.async_copy(h_hbm.at[sv1], rows1_v, sem_g1)
        weights(off0, sv0, dv0, w0_buf)
        weights(off1, sv1, dv1, w1_buf)
        g0.wait()
        scale(rows0_v, w0_buf)
        s0a = pltpu.async_copy(rows0_v, out_sh.at[dv0], sem_s, add=True)
        s0b = pltpu.async_copy(w0_buf, s_sh.at[dv0], sem_s, add=True)
        g1.wait()
        scale(rows1_v, w1_buf)
        s1a = pltpu.async_copy(rows1_v, out_sh.at[dv1], sem_s, add=True)
        s1b = pltpu.async_copy(w1_buf, s_sh.at[dv1], sem_s, add=True)
        s0a.wait()
        s0b.wait()
        s1a.wait()
        s1b.wait()

    set_a = (src0_v, dst0_v, src1_v, dst1_v)
    set_b = (src2_v, dst2_v, src3_v, dst3_v)
    fire_idx(0, *set_a)

    def two_pairs(q, _):
        pair(2 * q, set_a, set_b)
        pair(2 * q + 1, set_b, set_a)
        return 0

    lax.fori_loop(0, NB // 4, two_pairs, 0)
    drain_idx(*set_a)
    plsc.subcore_barrier()

    pltpu.sync_copy(out_sh.at[pl.ds(r0, ROWS_PER_TILE)],
                    msg_hbm.at[c, pl.ds(r0, ROWS_PER_TILE)])
    pltpu.sync_copy(s_sh.at[pl.ds(r0, ROWS_PER_TILE)],
                    sw_hbm.at[pl.ds(c * N_PAD + r0, ROWS_PER_TILE)])


def _stage2(h, src_p, dst_p, a_src, a_dst, mx):
    mesh = plsc.VectorSubcoreMesh(core_axis_name="c", subcore_axis_name="s")
    kern = functools.partial(
        pl.kernel,
        mesh=mesh,
        compiler_params=pltpu.CompilerParams(
            needs_layout_passes=False,
            use_tc_tiling_on_sc=False,
        ),
        out_type=[
            jax.ShapeDtypeStruct((NC, N_PAD, C), jnp.float32),
            jax.ShapeDtypeStruct((NC * N_PAD,), jnp.float32),
        ],
        scratch_types=[
            pltpu.VMEM((N,), jnp.float32),
            pltpu.VMEM((N,), jnp.float32),
            pltpu.VMEM((K,), jnp.int32),
            pltpu.VMEM((K,), jnp.int32),
            pltpu.VMEM((K,), jnp.int32),
            pltpu.VMEM((K,), jnp.int32),
            pltpu.VMEM((K,), jnp.int32),
            pltpu.VMEM((K,), jnp.int32),
            pltpu.VMEM((K,), jnp.int32),
            pltpu.VMEM((K,), jnp.int32),
            pltpu.VMEM((K, C), jnp.float32),
            pltpu.VMEM((K, C), jnp.float32),
            pltpu.VMEM((K,), jnp.float32),
            pltpu.VMEM((K,), jnp.float32),
            pltpu.VMEM((ROWS_PER_TILE,), jnp.float32),
            pltpu.VMEM((16,), jnp.float32),
            pltpu.VMEM_SHARED((N_PAD, C), jnp.float32),
            pltpu.VMEM_SHARED((N_PAD,), jnp.float32),
            pltpu.SemaphoreType.DMA,
            pltpu.SemaphoreType.DMA,
            pltpu.SemaphoreType.DMA,
            pltpu.SemaphoreType.DMA,
        ],
    )(_stage2_body)
    return kern(h, src_p, dst_p, a_src, a_dst, mx)


_BN3 = 400


def _stage3_body(msg_ref, sw_ref, bias_ref, out_ref):
    m = msg_ref[0] + msg_ref[1]
    s = sw_ref[0] + sw_ref[1]
    o = m / (s + 1e-16) + bias_ref[...]
    o = jnp.maximum(o, 0.0)
    nrm = jnp.sqrt(jnp.sum(o * o, axis=1, keepdims=True))
    nrm = jnp.maximum(nrm, 1e-12)
    out_ref[...] = o / nrm


def _stage3(msg_p, sw_p, bias):
    grid = (N // _BN3,)
    return pl.pallas_call(
        _stage3_body,
        grid=grid,
        in_specs=[
            pl.BlockSpec((NC, _BN3, C), lambda i: (0, i, 0)),
            pl.BlockSpec((NC, _BN3, 1), lambda i: (0, i, 0)),
            pl.BlockSpec((1, C), lambda i: (0, 0)),
        ],
        out_specs=pl.BlockSpec((_BN3, C), lambda i: (i, 0)),
        out_shape=jax.ShapeDtypeStruct((N, C), jnp.float32),
    )(msg_p, sw_p, bias)


def kernel(x, edge_index, W, att_src, att_dst, bias):
    h, a_src, a_dst, mx = _stage1(x, W, att_src, att_dst)

    loop = jnp.arange(N, dtype=edge_index.dtype)
    src = jnp.concatenate([edge_index[0], loop]).astype(jnp.int32)
    dst = jnp.concatenate([edge_index[1], loop]).astype(jnp.int32)
    src_p = jnp.pad(src, (0, E_PAD - E1 + 2 * K))
    dst_p = jnp.pad(dst, (0, E_PAD - E1 + 2 * K))

    msg_p, sw_flat = _stage2(h, src_p, dst_p,
                             a_src.reshape(N), a_dst.reshape(N),
                             mx.reshape(16))
    sw_p = sw_flat.reshape(NC, N_PAD, 1)

    return _stage3(msg_p, sw_p, bias.reshape(1, C))
